# Initial kernel scaffold; baseline (speedup 1.0000x reference)
import jax
import jax.numpy as jnp
from jax.experimental import pallas as pl


def kernel(hidden_c0, hidden_h0, hidden_c1, hidden_h1, instruction_pointer, current_step, node_embeddings, Wx0, Wh0, b0, Wx1, Wh1, b1, raise_kernel, raise_bias, branch_kernel, branch_bias, edge_sources, edge_dests, edge_types, true_indexes, false_indexes, exit_indexes, step_limits):
    raise NotImplementedError("write your pallas kernel here")


# speedup vs baseline: 2.3620x; 2.3620x over previous
import functools

import jax
import jax.numpy as jnp
from jax import lax
from jax.experimental import pallas as pl
from jax.experimental.pallas import tpu as pltpu
from jax.experimental.pallas import tpu_sc as plsc

B, N, H = 8, 1024, 256
D4 = 4 * H
M = 512
NB = (B * N) // M


def _dense_body(x_ref, c0_ref, h0_ref, c1_ref, h1_ref, ip_ref, keep_ref,
                ti_ref, fi_ref, wx0_ref, wh0_ref, b0_ref, wx1_ref, wh1_ref,
                b1_ref, k8_ref, kb_ref,
                st_ref, sf_ref, rsum_ref, wsum_ref, ipsc_ref):
    x = x_ref[...]
    c0 = c0_ref[...]
    h0 = h0_ref[...]
    c1 = c1_ref[...]
    h1 = h1_ref[...]
    f32 = jnp.float32

    z0 = (jnp.dot(x, wx0_ref[...], preferred_element_type=f32)
          + jnp.dot(h0, wh0_ref[...], preferred_element_type=f32)
          + b0_ref[...])
    i0 = jax.nn.sigmoid(z0[:, 0:H])
    f0 = jax.nn.sigmoid(z0[:, H:2 * H])
    g0 = jnp.tanh(z0[:, 2 * H:3 * H])
    o0 = jax.nn.sigmoid(z0[:, 3 * H:4 * H])
    c0n = f0 * c0 + i0 * g0
    h0n = o0 * jnp.tanh(c0n)

    z1 = (jnp.dot(h0n, wx1_ref[...], preferred_element_type=f32)
          + jnp.dot(h1, wh1_ref[...], preferred_element_type=f32)
          + b1_ref[...])
    i1 = jax.nn.sigmoid(z1[:, 0:H])
    f1 = jax.nn.sigmoid(z1[:, H:2 * H])
    g1 = jnp.tanh(z1[:, 2 * H:3 * H])
    o1 = jax.nn.sigmoid(z1[:, 3 * H:4 * H])
    c1n = f1 * c1 + i1 * g1
    h1n = o1 * jnp.tanh(c1n)

    keep = keep_ref[...]
    c0n = keep * c0n + (1.0 - keep) * c0
    h0n = keep * h0n + (1.0 - keep) * h0
    c1n = keep * c1n + (1.0 - keep) * c1
    h1n = keep * h1n + (1.0 - keep) * h1

    emb = jnp.concatenate([c0n, h0n, c1n, h1n], axis=1)

    lg = jnp.dot(emb, k8_ref[...], preferred_element_type=f32) + kb_ref[...]
    r0 = lg[:, 0:1]
    r1 = lg[:, 1:2]
    mr = jnp.maximum(r0, r1)
    e0 = jnp.exp(r0 - mr)
    e1 = jnp.exp(r1 - mr)
    sr = e0 + e1
    p_raise = e0 / sr
    p_nor = e1 / sr
    q0 = lg[:, 2:3]
    q1 = lg[:, 3:4]
    mq = jnp.maximum(q0, q1)
    d0 = jnp.exp(q0 - mq)
    d1 = jnp.exp(q1 - mq)
    sq = d0 + d1
    pb0 = d0 / sq
    pb1 = d1 / sq

    ip = ip_ref[...]
    wr = p_raise * ip
    wt = p_nor * pb0 * ip
    wf = p_nor * pb1 * ip

    st_ref[...] = emb * wt
    sf_ref[...] = emb * wf

    rp = lax.dot_general(wr, emb, (((0,), (0,)), ((), ())),
                         preferred_element_type=f32)
    wsp = jnp.sum(wr)
    dcol = lax.broadcasted_iota(jnp.int32, (M, D4), 1)
    tb = ti_ref[...]
    fb = fi_ref[...]
    ipp = (jnp.sum(jnp.where(tb == dcol, wt, 0.0), axis=0, keepdims=True)
           + jnp.sum(jnp.where(fb == dcol, wf, 0.0), axis=0, keepdims=True))

    first = pl.program_id(0) % 2 == 0

    @pl.when(first)
    def _():
        rsum_ref[...] = rp[None]
        wsum_ref[...] = jnp.zeros((1, 1, 128), f32) + wsp
        ipsc_ref[...] = ipp[None]

    @pl.when(jnp.logical_not(first))
    def _():
        rsum_ref[...] += rp[None]
        wsum_ref[...] += wsp
        ipsc_ref[...] += ipp[None]


def _dense(xf, c0f, h0f, c1f, h1f, ipc, keep, tif, fif,
           Wx0, Wh0, b0r, Wx1, Wh1, b1r, K8, kb):
    f32 = jnp.float32
    row = lambda w: pl.BlockSpec((M, w), lambda i: (i, 0))
    full = lambda a, b: pl.BlockSpec((a, b), lambda i: (0, 0))
    acc = lambda w: pl.BlockSpec((1, 1, w), lambda i: (i // 2, 0, 0))
    return pl.pallas_call(
        _dense_body,
        grid=(NB,),
        in_specs=[row(H), row(H), row(H), row(H), row(H), row(1), row(1),
                  row(1), row(1), full(H, D4), full(H, D4), full(1, D4),
                  full(H, D4), full(H, D4), full(1, D4), full(D4, 8),
                  full(1, 8)],
        out_specs=[row(D4), row(D4), acc(D4), acc(128), acc(D4)],
        out_shape=[jax.ShapeDtypeStruct((B * N, D4), f32),
                   jax.ShapeDtypeStruct((B * N, D4), f32),
                   jax.ShapeDtypeStruct((B, 1, D4), f32),
                   jax.ShapeDtypeStruct((B, 1, 128), f32),
                   jax.ShapeDtypeStruct((B, 1, D4), f32)],
    )(xf, c0f, h0f, c1f, h1f, ipc, keep, tif, fif,
      Wx0, Wh0, b0r, Wx1, Wh1, b1r, K8, kb)


_NC, _NS = 2, 16
_NW = _NC * _NS
_CW = 128
_CG = D4 // _CW
_NH = 2
_RH = N // _NH
_CH = 128
_NT = B * _CG * _NH
_RNDS = _NT // _NW


def _sc_body(st_hbm, sf_hbm, ti_hbm, fi_hbm, zeros_hbm, out_hbm,
             st_v, sf_v, ti_v, fi_v, accum):
    c = lax.axis_index("c")
    s = lax.axis_index("s")
    wid = s * _NC + c

    def round_body(r, carry0):
        t = r * _NW + wid
        b = t // (_CG * _NH)
        rem = t % (_CG * _NH)
        g = rem // _NH
        h = rem % _NH
        row_base = b * N
        col0 = g * _CW
        lo = h * _RH
        pltpu.sync_copy(zeros_hbm, accum)

        def chunk_body(ch, carry1):
            row0 = row_base + ch * _CH
            pltpu.sync_copy(ti_hbm.at[pl.ds(row0, _CH)], ti_v)
            pltpu.sync_copy(fi_hbm.at[pl.ds(row0, _CH)], fi_v)
            pltpu.sync_copy(st_hbm.at[pl.ds(row0, _CH), pl.ds(col0, _CW)],
                            st_v)
            pltpu.sync_copy(sf_hbm.at[pl.ds(row0, _CH), pl.ds(col0, _CW)],
                            sf_v)

            def group_body(gi, carry2):
                tv = ti_v[pl.ds(gi * 16, 16)]
                fv = fi_v[pl.ds(gi * 16, 16)]
                for k in range(16):
                    row = gi * 16 + k
                    dt = tv[k]
                    df = fv[k]
                    lt = dt - lo
                    lf = df - lo

                    @pl.when(jnp.logical_and(dt >= lo, dt < lo + _RH))
                    def _():
                        for j in range(_CW // 16):
                            sl = pl.ds(j * 16, 16)
                            accum[lt, sl] = accum[lt, sl] + st_v[row, sl]

                    @pl.when(jnp.logical_and(df >= lo, df < lo + _RH))
                    def _():
                        for j in range(_CW // 16):
                            sl = pl.ds(j * 16, 16)
                            accum[lf, sl] = accum[lf, sl] + sf_v[row, sl]
                return carry2

            lax.fori_loop(0, _CH // 16, group_body, 0)
            return carry1

        lax.fori_loop(0, N // _CH, chunk_body, 0)
        pltpu.sync_copy(accum,
                        out_hbm.at[pl.ds(row_base + lo, _RH),
                                   pl.ds(col0, _CW)])
        return carry0

    lax.fori_loop(0, _RNDS, round_body, 0)


@functools.lru_cache(maxsize=1)
def _build_sc_scatter():
    return functools.partial(
        pl.kernel,
        out_type=jax.ShapeDtypeStruct((B * N, D4), jnp.float32),
        mesh=plsc.VectorSubcoreMesh(core_axis_name="c", subcore_axis_name="s"),
        scratch_types=[
            pltpu.VMEM((_CH, _CW), jnp.float32),
            pltpu.VMEM((_CH, _CW), jnp.float32),
            pltpu.VMEM((_CH,), jnp.int32),
            pltpu.VMEM((_CH,), jnp.int32),
            pltpu.VMEM((_RH, _CW), jnp.float32),
        ],
    )(_sc_body)


def _final_body(agg_ref, c0_ref, h0_ref, c1_ref, h1_ref, ipsc_ref, ipold_ref,
                live_ref, exf_ref, rsum_ref, wsum_ref, hid_ref, ipo_ref):
    exf = exf_ref[...]
    agg = agg_ref[...] + exf * rsum_ref[0]
    w0 = wsum_ref[0][0:1, 0:1]
    ipn = ipsc_ref[...] + exf * w0
    denom = ipn + 1e-7
    live = live_ref[...]
    cat_old = jnp.concatenate(
        [c0_ref[...], h0_ref[...], c1_ref[...], h1_ref[...]], axis=1)
    hid_ref[...] = live * (agg / denom) + (1.0 - live) * cat_old
    ipo_ref[...] = live * ipn + (1.0 - live) * ipold_ref[...]


def _final(agg, c0f, h0f, c1f, h1f, ipscf, ipold, livef, exf, rsum, wsum):
    f32 = jnp.float32
    row = lambda w: pl.BlockSpec((M, w), lambda i: (i, 0))
    per_b = lambda w: pl.BlockSpec((1, 1, w), lambda i: (i // 2, 0, 0))
    return pl.pallas_call(
        _final_body,
        grid=(NB,),
        in_specs=[row(D4), row(H), row(H), row(H), row(H), row(1), row(1),
                  row(1), row(1), per_b(D4), per_b(128)],
        out_specs=[row(D4), row(1)],
        out_shape=[jax.ShapeDtypeStruct((B * N, D4), f32),
                   jax.ShapeDtypeStruct((B * N, 1), f32)],
    )(agg, c0f, h0f, c1f, h1f, ipscf, ipold, livef, exf, rsum, wsum)


def kernel(hidden_c0, hidden_h0, hidden_c1, hidden_h1, instruction_pointer,
           current_step, node_embeddings, Wx0, Wh0, b0, Wx1, Wh1, b1,
           raise_kernel, raise_bias, branch_kernel, branch_bias,
           edge_sources, edge_dests, edge_types, true_indexes, false_indexes,
           exit_indexes, step_limits):
    f32 = jnp.float32
    xf = node_embeddings.reshape(B * N, H)
    c0f = hidden_c0.reshape(B * N, H)
    h0f = hidden_h0.reshape(B * N, H)
    c1f = hidden_c1.reshape(B * N, H)
    h1f = hidden_h1.reshape(B * N, H)
    ipc = instruction_pointer.reshape(B * N, 1)
    tif = true_indexes.reshape(B * N, 1)
    fif = false_indexes.reshape(B * N, 1)

    node_ids = jnp.arange(N, dtype=jnp.int32)[None, :]
    is_exit = node_ids == exit_indexes[:, None].astype(jnp.int32)
    keep = jnp.where(is_exit, 0.0, 1.0).astype(f32).reshape(B * N, 1)
    exf = jnp.where(is_exit, 1.0, 0.0).astype(f32).reshape(B * N, 1)
    live = (current_step < step_limits).astype(f32)
    livef = jnp.broadcast_to(live[:, None], (B, N)).reshape(B * N, 1)

    K8 = jnp.concatenate(
        [raise_kernel, branch_kernel, jnp.zeros((D4, 4), f32)], axis=1)
    kb = jnp.concatenate(
        [raise_bias, branch_bias, jnp.zeros((4,), f32)])[None, :]
    b0r = b0[None, :]
    b1r = b1[None, :]

    st, sf, rsum, wsum, ipsc = _dense(
        xf, c0f, h0f, c1f, h1f, ipc, keep, tif, fif,
        Wx0, Wh0, b0r, Wx1, Wh1, b1r, K8, kb)

    zeros64 = jnp.zeros((_RH, _CW), f32)
    agg = _build_sc_scatter()(st, sf, true_indexes.reshape(B * N),
                              false_indexes.reshape(B * N), zeros64)

    ipscf = ipsc.reshape(B * N, 1)
    hid, ipo = _final(agg, c0f, h0f, c1f, h1f, ipscf, ipc, livef, exf,
                      rsum, wsum)
    return hid.reshape(B, N, D4), ipo.reshape(B, N)


# speedup vs baseline: 3.5001x; 1.4818x over previous
import functools

import jax
import jax.numpy as jnp
from jax import lax
from jax.experimental import pallas as pl
from jax.experimental.pallas import tpu as pltpu
from jax.experimental.pallas import tpu_sc as plsc

B, N, H = 8, 1024, 256
D4 = 4 * H
M = 512
NB = (B * N) // M


def _dense_body(x_ref, c0_ref, h0_ref, c1_ref, h1_ref, ip_ref, keep_ref,
                ti_ref, fi_ref, wx0_ref, wh0_ref, b0_ref, wx1_ref, wh1_ref,
                b1_ref, k8_ref, kb_ref,
                st_ref, sf_ref, rsum_ref, wsum_ref, ipsc_ref):
    x = x_ref[...]
    c0 = c0_ref[...]
    h0 = h0_ref[...]
    c1 = c1_ref[...]
    h1 = h1_ref[...]
    f32 = jnp.float32

    z0 = (jnp.dot(x, wx0_ref[...], preferred_element_type=f32)
          + jnp.dot(h0, wh0_ref[...], preferred_element_type=f32)
          + b0_ref[...])
    i0 = jax.nn.sigmoid(z0[:, 0:H])
    f0 = jax.nn.sigmoid(z0[:, H:2 * H])
    g0 = jnp.tanh(z0[:, 2 * H:3 * H])
    o0 = jax.nn.sigmoid(z0[:, 3 * H:4 * H])
    c0n = f0 * c0 + i0 * g0
    h0n = o0 * jnp.tanh(c0n)

    z1 = (jnp.dot(h0n, wx1_ref[...], preferred_element_type=f32)
          + jnp.dot(h1, wh1_ref[...], preferred_element_type=f32)
          + b1_ref[...])
    i1 = jax.nn.sigmoid(z1[:, 0:H])
    f1 = jax.nn.sigmoid(z1[:, H:2 * H])
    g1 = jnp.tanh(z1[:, 2 * H:3 * H])
    o1 = jax.nn.sigmoid(z1[:, 3 * H:4 * H])
    c1n = f1 * c1 + i1 * g1
    h1n = o1 * jnp.tanh(c1n)

    keep = keep_ref[...]
    c0n = keep * c0n + (1.0 - keep) * c0
    h0n = keep * h0n + (1.0 - keep) * h0
    c1n = keep * c1n + (1.0 - keep) * c1
    h1n = keep * h1n + (1.0 - keep) * h1

    emb = jnp.concatenate([c0n, h0n, c1n, h1n], axis=1)

    lg = jnp.dot(emb, k8_ref[...], preferred_element_type=f32) + kb_ref[...]
    r0 = lg[:, 0:1]
    r1 = lg[:, 1:2]
    mr = jnp.maximum(r0, r1)
    e0 = jnp.exp(r0 - mr)
    e1 = jnp.exp(r1 - mr)
    sr = e0 + e1
    p_raise = e0 / sr
    p_nor = e1 / sr
    q0 = lg[:, 2:3]
    q1 = lg[:, 3:4]
    mq = jnp.maximum(q0, q1)
    d0 = jnp.exp(q0 - mq)
    d1 = jnp.exp(q1 - mq)
    sq = d0 + d1
    pb0 = d0 / sq
    pb1 = d1 / sq

    ip = ip_ref[...]
    wr = p_raise * ip
    wt = p_nor * pb0 * ip
    wf = p_nor * pb1 * ip

    st_ref[...] = emb * wt
    sf_ref[...] = emb * wf

    rp = lax.dot_general(wr, emb, (((0,), (0,)), ((), ())),
                         preferred_element_type=f32)
    wsp = jnp.sum(wr)
    dcol = lax.broadcasted_iota(jnp.int32, (M, D4), 1)
    tb = ti_ref[...]
    fb = fi_ref[...]
    ipp = (jnp.sum(jnp.where(tb == dcol, wt, 0.0), axis=0, keepdims=True)
           + jnp.sum(jnp.where(fb == dcol, wf, 0.0), axis=0, keepdims=True))

    first = pl.program_id(0) % 2 == 0

    @pl.when(first)
    def _():
        rsum_ref[...] = rp[None]
        wsum_ref[...] = jnp.zeros((1, 1, 128), f32) + wsp
        ipsc_ref[...] = ipp[None]

    @pl.when(jnp.logical_not(first))
    def _():
        rsum_ref[...] += rp[None]
        wsum_ref[...] += wsp
        ipsc_ref[...] += ipp[None]


def _dense(xf, c0f, h0f, c1f, h1f, ipc, keep, tif, fif,
           Wx0, Wh0, b0r, Wx1, Wh1, b1r, K8, kb):
    f32 = jnp.float32
    row = lambda w: pl.BlockSpec((M, w), lambda i: (i, 0))
    full = lambda a, b: pl.BlockSpec((a, b), lambda i: (0, 0))
    acc = lambda w: pl.BlockSpec((1, 1, w), lambda i: (i // 2, 0, 0))
    return pl.pallas_call(
        _dense_body,
        grid=(NB,),
        in_specs=[row(H), row(H), row(H), row(H), row(H), row(1), row(1),
                  row(1), row(1), full(H, D4), full(H, D4), full(1, D4),
                  full(H, D4), full(H, D4), full(1, D4), full(D4, 8),
                  full(1, 8)],
        out_specs=[row(D4), row(D4), acc(D4), acc(128), acc(D4)],
        out_shape=[jax.ShapeDtypeStruct((B * N, D4), f32),
                   jax.ShapeDtypeStruct((B * N, D4), f32),
                   jax.ShapeDtypeStruct((B, 1, D4), f32),
                   jax.ShapeDtypeStruct((B, 1, 128), f32),
                   jax.ShapeDtypeStruct((B, 1, D4), f32)],
    )(xf, c0f, h0f, c1f, h1f, ipc, keep, tif, fif,
      Wx0, Wh0, b0r, Wx1, Wh1, b1r, K8, kb)


_NC, _NS = 2, 16
_NW = _NC * _NS
_CW = 128
_CG = D4 // _CW
_NH = 2
_RH = N // _NH
_CH = 128
_NT = B * _CG * _NH
_RNDS = _NT // _NW


def _sc_body(st_hbm, sf_hbm, ti_hbm, fi_hbm, zeros_hbm, out_hbm,
             st_v, sf_v, ti_v, fi_v, accum):
    c = lax.axis_index("c")
    s = lax.axis_index("s")
    wid = s * _NC + c

    def round_body(r, carry0):
        t = r * _NW + wid
        b = t // (_CG * _NH)
        rem = t % (_CG * _NH)
        g = rem // _NH
        h = rem % _NH
        row_base = b * N
        col0 = g * _CW
        lo = h * _RH
        pltpu.sync_copy(zeros_hbm, accum)

        def chunk_body(ch, carry1):
            row0 = row_base + ch * _CH
            pltpu.sync_copy(ti_hbm.at[pl.ds(row0, _CH)], ti_v)
            pltpu.sync_copy(fi_hbm.at[pl.ds(row0, _CH)], fi_v)
            pltpu.sync_copy(st_hbm.at[pl.ds(row0, _CH), pl.ds(col0, _CW)],
                            st_v)
            pltpu.sync_copy(sf_hbm.at[pl.ds(row0, _CH), pl.ds(col0, _CW)],
                            sf_v)

            def group_body(gi, carry2):
                tv = ti_v[pl.ds(gi * 16, 16)]
                fv = fi_v[pl.ds(gi * 16, 16)]
                for k in range(16):
                    row = gi * 16 + k
                    dt = tv[k]
                    df = fv[k]
                    lt = dt - lo
                    lf = df - lo

                    @pl.when(jnp.logical_and(dt >= lo, dt < lo + _RH))
                    def _():
                        nj = _CW // 16
                        sls = [pl.ds(j * 16, 16) for j in range(nj)]
                        acc = [accum[lt, sls[j]] for j in range(nj)]
                        src = [st_v[row, sls[j]] for j in range(nj)]
                        for j in range(nj):
                            accum[lt, sls[j]] = acc[j] + src[j]

                    @pl.when(jnp.logical_and(df >= lo, df < lo + _RH))
                    def _():
                        nj = _CW // 16
                        sls = [pl.ds(j * 16, 16) for j in range(nj)]
                        acc = [accum[lf, sls[j]] for j in range(nj)]
                        src = [sf_v[row, sls[j]] for j in range(nj)]
                        for j in range(nj):
                            accum[lf, sls[j]] = acc[j] + src[j]
                return carry2

            lax.fori_loop(0, _CH // 16, group_body, 0)
            return carry1

        lax.fori_loop(0, N // _CH, chunk_body, 0)
        pltpu.sync_copy(accum,
                        out_hbm.at[pl.ds(row_base + lo, _RH),
                                   pl.ds(col0, _CW)])
        return carry0

    lax.fori_loop(0, _RNDS, round_body, 0)


@functools.lru_cache(maxsize=1)
def _build_sc_scatter():
    return functools.partial(
        pl.kernel,
        out_type=jax.ShapeDtypeStruct((B * N, D4), jnp.float32),
        mesh=plsc.VectorSubcoreMesh(core_axis_name="c", subcore_axis_name="s"),
        scratch_types=[
            pltpu.VMEM((_CH, _CW), jnp.float32),
            pltpu.VMEM((_CH, _CW), jnp.float32),
            pltpu.VMEM((_CH,), jnp.int32),
            pltpu.VMEM((_CH,), jnp.int32),
            pltpu.VMEM((_RH, _CW), jnp.float32),
        ],
    )(_sc_body)


def _final_body(agg_ref, c0_ref, h0_ref, c1_ref, h1_ref, ipsc_ref, ipold_ref,
                live_ref, exf_ref, rsum_ref, wsum_ref, hid_ref, ipo_ref):
    exf = exf_ref[...]
    agg = agg_ref[...] + exf * rsum_ref[0]
    w0 = wsum_ref[0][0:1, 0:1]
    ipn = ipsc_ref[...] + exf * w0
    denom = ipn + 1e-7
    live = live_ref[...]
    cat_old = jnp.concatenate(
        [c0_ref[...], h0_ref[...], c1_ref[...], h1_ref[...]], axis=1)
    hid_ref[...] = live * (agg / denom) + (1.0 - live) * cat_old
    ipo_ref[...] = live * ipn + (1.0 - live) * ipold_ref[...]


def _final(agg, c0f, h0f, c1f, h1f, ipscf, ipold, livef, exf, rsum, wsum):
    f32 = jnp.float32
    row = lambda w: pl.BlockSpec((M, w), lambda i: (i, 0))
    per_b = lambda w: pl.BlockSpec((1, 1, w), lambda i: (i // 2, 0, 0))
    return pl.pallas_call(
        _final_body,
        grid=(NB,),
        in_specs=[row(D4), row(H), row(H), row(H), row(H), row(1), row(1),
                  row(1), row(1), per_b(D4), per_b(128)],
        out_specs=[row(D4), row(1)],
        out_shape=[jax.ShapeDtypeStruct((B * N, D4), f32),
                   jax.ShapeDtypeStruct((B * N, 1), f32)],
    )(agg, c0f, h0f, c1f, h1f, ipscf, ipold, livef, exf, rsum, wsum)


def kernel(hidden_c0, hidden_h0, hidden_c1, hidden_h1, instruction_pointer,
           current_step, node_embeddings, Wx0, Wh0, b0, Wx1, Wh1, b1,
           raise_kernel, raise_bias, branch_kernel, branch_bias,
           edge_sources, edge_dests, edge_types, true_indexes, false_indexes,
           exit_indexes, step_limits):
    f32 = jnp.float32
    xf = node_embeddings.reshape(B * N, H)
    c0f = hidden_c0.reshape(B * N, H)
    h0f = hidden_h0.reshape(B * N, H)
    c1f = hidden_c1.reshape(B * N, H)
    h1f = hidden_h1.reshape(B * N, H)
    ipc = instruction_pointer.reshape(B * N, 1)
    tif = true_indexes.reshape(B * N, 1)
    fif = false_indexes.reshape(B * N, 1)

    node_ids = jnp.arange(N, dtype=jnp.int32)[None, :]
    is_exit = node_ids == exit_indexes[:, None].astype(jnp.int32)
    keep = jnp.where(is_exit, 0.0, 1.0).astype(f32).reshape(B * N, 1)
    exf = jnp.where(is_exit, 1.0, 0.0).astype(f32).reshape(B * N, 1)
    live = (current_step < step_limits).astype(f32)
    livef = jnp.broadcast_to(live[:, None], (B, N)).reshape(B * N, 1)

    K8 = jnp.concatenate(
        [raise_kernel, branch_kernel, jnp.zeros((D4, 4), f32)], axis=1)
    kb = jnp.concatenate(
        [raise_bias, branch_bias, jnp.zeros((4,), f32)])[None, :]
    b0r = b0[None, :]
    b1r = b1[None, :]

    st, sf, rsum, wsum, ipsc = _dense(
        xf, c0f, h0f, c1f, h1f, ipc, keep, tif, fif,
        Wx0, Wh0, b0r, Wx1, Wh1, b1r, K8, kb)

    zeros64 = jnp.zeros((_RH, _CW), f32)
    agg = _build_sc_scatter()(st, sf, true_indexes.reshape(B * N),
                              false_indexes.reshape(B * N), zeros64)

    ipscf = ipsc.reshape(B * N, 1)
    hid, ipo = _final(agg, c0f, h0f, c1f, h1f, ipscf, ipc, livef, exf,
                      rsum, wsum)
    return hid.reshape(B, N, D4), ipo.reshape(B, N)


# speedup vs baseline: 4.3820x; 1.2520x over previous
import functools

import jax
import jax.numpy as jnp
from jax import lax
from jax.experimental import pallas as pl
from jax.experimental.pallas import tpu as pltpu
from jax.experimental.pallas import tpu_sc as plsc

B, N, H = 8, 1024, 256
D4 = 4 * H
M = 512
NB = (B * N) // M


def _dense_body(x_ref, c0_ref, h0_ref, c1_ref, h1_ref, ip_ref, keep_ref,
                ti_ref, fi_ref, wx0_ref, wh0_ref, b0_ref, wx1_ref, wh1_ref,
                b1_ref, k8_ref, kb_ref,
                st_ref, sf_ref, rsum_ref, wsum_ref, ipsc_ref):
    x = x_ref[...]
    c0 = c0_ref[...]
    h0 = h0_ref[...]
    c1 = c1_ref[...]
    h1 = h1_ref[...]
    f32 = jnp.float32

    z0 = (jnp.dot(x, wx0_ref[...], preferred_element_type=f32)
          + jnp.dot(h0, wh0_ref[...], preferred_element_type=f32)
          + b0_ref[...])
    i0 = jax.nn.sigmoid(z0[:, 0:H])
    f0 = jax.nn.sigmoid(z0[:, H:2 * H])
    g0 = jnp.tanh(z0[:, 2 * H:3 * H])
    o0 = jax.nn.sigmoid(z0[:, 3 * H:4 * H])
    c0n = f0 * c0 + i0 * g0
    h0n = o0 * jnp.tanh(c0n)

    z1 = (jnp.dot(h0n, wx1_ref[...], preferred_element_type=f32)
          + jnp.dot(h1, wh1_ref[...], preferred_element_type=f32)
          + b1_ref[...])
    i1 = jax.nn.sigmoid(z1[:, 0:H])
    f1 = jax.nn.sigmoid(z1[:, H:2 * H])
    g1 = jnp.tanh(z1[:, 2 * H:3 * H])
    o1 = jax.nn.sigmoid(z1[:, 3 * H:4 * H])
    c1n = f1 * c1 + i1 * g1
    h1n = o1 * jnp.tanh(c1n)

    keep = keep_ref[...]
    c0n = keep * c0n + (1.0 - keep) * c0
    h0n = keep * h0n + (1.0 - keep) * h0
    c1n = keep * c1n + (1.0 - keep) * c1
    h1n = keep * h1n + (1.0 - keep) * h1

    emb = jnp.concatenate([c0n, h0n, c1n, h1n], axis=1)

    lg = jnp.dot(emb, k8_ref[...], preferred_element_type=f32) + kb_ref[...]
    r0 = lg[:, 0:1]
    r1 = lg[:, 1:2]
    mr = jnp.maximum(r0, r1)
    e0 = jnp.exp(r0 - mr)
    e1 = jnp.exp(r1 - mr)
    sr = e0 + e1
    p_raise = e0 / sr
    p_nor = e1 / sr
    q0 = lg[:, 2:3]
    q1 = lg[:, 3:4]
    mq = jnp.maximum(q0, q1)
    d0 = jnp.exp(q0 - mq)
    d1 = jnp.exp(q1 - mq)
    sq = d0 + d1
    pb0 = d0 / sq
    pb1 = d1 / sq

    ip = ip_ref[...]
    wr = p_raise * ip
    wt = p_nor * pb0 * ip
    wf = p_nor * pb1 * ip

    st_ref[...] = emb * wt
    sf_ref[...] = emb * wf

    rp = lax.dot_general(wr, emb, (((0,), (0,)), ((), ())),
                         preferred_element_type=f32)
    wsp = jnp.sum(wr)
    dcol = lax.broadcasted_iota(jnp.int32, (M, D4), 1)
    tb = ti_ref[...]
    fb = fi_ref[...]
    ipp = (jnp.sum(jnp.where(tb == dcol, wt, 0.0), axis=0, keepdims=True)
           + jnp.sum(jnp.where(fb == dcol, wf, 0.0), axis=0, keepdims=True))

    first = pl.program_id(0) % 2 == 0

    @pl.when(first)
    def _():
        rsum_ref[...] = rp[None]
        wsum_ref[...] = jnp.zeros((1, 1, 128), f32) + wsp
        ipsc_ref[...] = ipp[None]

    @pl.when(jnp.logical_not(first))
    def _():
        rsum_ref[...] += rp[None]
        wsum_ref[...] += wsp
        ipsc_ref[...] += ipp[None]


def _dense(xf, c0f, h0f, c1f, h1f, ipc, keep, tif, fif,
           Wx0, Wh0, b0r, Wx1, Wh1, b1r, K8, kb):
    f32 = jnp.float32
    row = lambda w: pl.BlockSpec((M, w), lambda i: (i, 0))
    full = lambda a, b: pl.BlockSpec((a, b), lambda i: (0, 0))
    acc = lambda w: pl.BlockSpec((1, 1, w), lambda i: (i // 2, 0, 0))
    return pl.pallas_call(
        _dense_body,
        grid=(NB,),
        in_specs=[row(H), row(H), row(H), row(H), row(H), row(1), row(1),
                  row(1), row(1), full(H, D4), full(H, D4), full(1, D4),
                  full(H, D4), full(H, D4), full(1, D4), full(D4, 8),
                  full(1, 8)],
        out_specs=[row(D4), row(D4), acc(D4), acc(128), acc(D4)],
        out_shape=[jax.ShapeDtypeStruct((B * N, D4), f32),
                   jax.ShapeDtypeStruct((B * N, D4), f32),
                   jax.ShapeDtypeStruct((B, 1, D4), f32),
                   jax.ShapeDtypeStruct((B, 1, 128), f32),
                   jax.ShapeDtypeStruct((B, 1, D4), f32)],
    )(xf, c0f, h0f, c1f, h1f, ipc, keep, tif, fif,
      Wx0, Wh0, b0r, Wx1, Wh1, b1r, K8, kb)


_NC, _NS = 2, 16
_NW = _NC * _NS
_CW = 128
_CG = D4 // _CW
_NH = 2
_RH = N // _NH
_CH = 64
_CPR = N // _CH
_NT = B * _CG * _NH
_RNDS = _NT // _NW
_GC = _RNDS * _CPR


def _sc_body(st_hbm, sf_hbm, ti_hbm, fi_hbm, zeros_hbm, out_hbm,
             stA, sfA, tiA, fiA, stB, sfB, tiB, fiB, semA, semB, accum):
    c = lax.axis_index("c")
    s = lax.axis_index("s")
    wid = s * _NC + c
    banks = ((stA, sfA, tiA, fiA, semA), (stB, sfB, tiB, fiB, semB))

    def task(r):
        t = r * _NW + wid
        b = t // (_CG * _NH)
        rem = t % (_CG * _NH)
        g = rem // _NH
        h = rem % _NH
        return b * N, g * _CW, h * _RH

    def descs(gc, bnk):
        r = gc // _CPR
        ci = gc % _CPR
        row_base, col0, _ = task(r)
        row0 = row_base + ci * _CH
        stb, sfb, tib, fib, sem = banks[bnk]
        return (
            pltpu.make_async_copy(ti_hbm.at[pl.ds(row0, _CH)], tib, sem),
            pltpu.make_async_copy(fi_hbm.at[pl.ds(row0, _CH)], fib, sem),
            pltpu.make_async_copy(
                st_hbm.at[pl.ds(row0, _CH), pl.ds(col0, _CW)], stb, sem),
            pltpu.make_async_copy(
                sf_hbm.at[pl.ds(row0, _CH), pl.ds(col0, _CW)], sfb, sem),
        )

    def fire(gc, bnk):
        for d in descs(gc, bnk):
            d.start()

    def wait(gc, bnk):
        for d in descs(gc, bnk):
            d.wait()

    pltpu.sync_copy(zeros_hbm, accum)
    fire(0, 0)

    def pair_body(gg, carry0):
        for bnk in range(2):
            g = gg * 2 + bnk
            _, _, lo = task(g // _CPR)

            @pl.when(g + 1 < _GC)
            def _():
                fire(g + 1, bnk ^ 1)

            wait(g, bnk)
            st_v, sf_v, ti_v, fi_v, _sem = banks[bnk]

            def group_body(gi, carry2):
                tv = ti_v[pl.ds(gi * 16, 16)]
                fv = fi_v[pl.ds(gi * 16, 16)]
                for k in range(16):
                    row = gi * 16 + k
                    dt = tv[k]
                    df = fv[k]
                    lt = dt - lo
                    lf = df - lo

                    @pl.when(jnp.logical_and(dt >= lo, dt < lo + _RH))
                    def _():
                        nj = _CW // 16
                        sls = [pl.ds(j * 16, 16) for j in range(nj)]
                        acc = [accum[lt, sls[j]] for j in range(nj)]
                        src = [st_v[row, sls[j]] for j in range(nj)]
                        for j in range(nj):
                            accum[lt, sls[j]] = acc[j] + src[j]

                    @pl.when(jnp.logical_and(df >= lo, df < lo + _RH))
                    def _():
                        nj = _CW // 16
                        sls = [pl.ds(j * 16, 16) for j in range(nj)]
                        acc = [accum[lf, sls[j]] for j in range(nj)]
                        src = [sf_v[row, sls[j]] for j in range(nj)]
                        for j in range(nj):
                            accum[lf, sls[j]] = acc[j] + src[j]
                return carry2

            lax.fori_loop(0, _CH // 16, group_body, 0)

            @pl.when((g % _CPR) == (_CPR - 1))
            def _():
                r = g // _CPR
                row_base, col0, lo2 = task(r)
                pltpu.sync_copy(accum,
                                out_hbm.at[pl.ds(row_base + lo2, _RH),
                                           pl.ds(col0, _CW)])
                pltpu.sync_copy(zeros_hbm, accum)
        return carry0

    lax.fori_loop(0, _GC // 2, pair_body, 0)


@functools.lru_cache(maxsize=1)
def _build_sc_scatter():
    return functools.partial(
        pl.kernel,
        out_type=jax.ShapeDtypeStruct((B * N, D4), jnp.float32),
        mesh=plsc.VectorSubcoreMesh(core_axis_name="c", subcore_axis_name="s"),
        scratch_types=[
            pltpu.VMEM((_CH, _CW), jnp.float32),
            pltpu.VMEM((_CH, _CW), jnp.float32),
            pltpu.VMEM((_CH,), jnp.int32),
            pltpu.VMEM((_CH,), jnp.int32),
            pltpu.VMEM((_CH, _CW), jnp.float32),
            pltpu.VMEM((_CH, _CW), jnp.float32),
            pltpu.VMEM((_CH,), jnp.int32),
            pltpu.VMEM((_CH,), jnp.int32),
            pltpu.SemaphoreType.DMA,
            pltpu.SemaphoreType.DMA,
            pltpu.VMEM((_RH, _CW), jnp.float32),
        ],
    )(_sc_body)


def _final_body(agg_ref, c0_ref, h0_ref, c1_ref, h1_ref, ipsc_ref, ipold_ref,
                live_ref, exf_ref, rsum_ref, wsum_ref, hid_ref, ipo_ref):
    exf = exf_ref[...]
    agg = agg_ref[...] + exf * rsum_ref[0]
    w0 = wsum_ref[0][0:1, 0:1]
    ipn = ipsc_ref[...] + exf * w0
    denom = ipn + 1e-7
    live = live_ref[...]
    cat_old = jnp.concatenate(
        [c0_ref[...], h0_ref[...], c1_ref[...], h1_ref[...]], axis=1)
    hid_ref[...] = live * (agg / denom) + (1.0 - live) * cat_old
    ipo_ref[...] = live * ipn + (1.0 - live) * ipold_ref[...]


def _final(agg, c0f, h0f, c1f, h1f, ipscf, ipold, livef, exf, rsum, wsum):
    f32 = jnp.float32
    row = lambda w: pl.BlockSpec((M, w), lambda i: (i, 0))
    per_b = lambda w: pl.BlockSpec((1, 1, w), lambda i: (i // 2, 0, 0))
    return pl.pallas_call(
        _final_body,
        grid=(NB,),
        in_specs=[row(D4), row(H), row(H), row(H), row(H), row(1), row(1),
                  row(1), row(1), per_b(D4), per_b(128)],
        out_specs=[row(D4), row(1)],
        out_shape=[jax.ShapeDtypeStruct((B * N, D4), f32),
                   jax.ShapeDtypeStruct((B * N, 1), f32)],
    )(agg, c0f, h0f, c1f, h1f, ipscf, ipold, livef, exf, rsum, wsum)


def kernel(hidden_c0, hidden_h0, hidden_c1, hidden_h1, instruction_pointer,
           current_step, node_embeddings, Wx0, Wh0, b0, Wx1, Wh1, b1,
           raise_kernel, raise_bias, branch_kernel, branch_bias,
           edge_sources, edge_dests, edge_types, true_indexes, false_indexes,
           exit_indexes, step_limits):
    f32 = jnp.float32
    xf = node_embeddings.reshape(B * N, H)
    c0f = hidden_c0.reshape(B * N, H)
    h0f = hidden_h0.reshape(B * N, H)
    c1f = hidden_c1.reshape(B * N, H)
    h1f = hidden_h1.reshape(B * N, H)
    ipc = instruction_pointer.reshape(B * N, 1)
    tif = true_indexes.reshape(B * N, 1)
    fif = false_indexes.reshape(B * N, 1)

    node_ids = jnp.arange(N, dtype=jnp.int32)[None, :]
    is_exit = node_ids == exit_indexes[:, None].astype(jnp.int32)
    keep = jnp.where(is_exit, 0.0, 1.0).astype(f32).reshape(B * N, 1)
    exf = jnp.where(is_exit, 1.0, 0.0).astype(f32).reshape(B * N, 1)
    live = (current_step < step_limits).astype(f32)
    livef = jnp.broadcast_to(live[:, None], (B, N)).reshape(B * N, 1)

    K8 = jnp.concatenate(
        [raise_kernel, branch_kernel, jnp.zeros((D4, 4), f32)], axis=1)
    kb = jnp.concatenate(
        [raise_bias, branch_bias, jnp.zeros((4,), f32)])[None, :]
    b0r = b0[None, :]
    b1r = b1[None, :]

    st, sf, rsum, wsum, ipsc = _dense(
        xf, c0f, h0f, c1f, h1f, ipc, keep, tif, fif,
        Wx0, Wh0, b0r, Wx1, Wh1, b1r, K8, kb)

    zeros64 = jnp.zeros((_RH, _CW), f32)
    agg = _build_sc_scatter()(st, sf, true_indexes.reshape(B * N),
                              false_indexes.reshape(B * N), zeros64)

    ipscf = ipsc.reshape(B * N, 1)
    hid, ipo = _final(agg, c0f, h0f, c1f, h1f, ipscf, ipc, livef, exf,
                      rsum, wsum)
    return hid.reshape(B, N, D4), ipo.reshape(B, N)


# speedup vs baseline: 4.6917x; 1.0707x over previous
import functools

import jax
import jax.numpy as jnp
from jax import lax
from jax.experimental import pallas as pl
from jax.experimental.pallas import tpu as pltpu
from jax.experimental.pallas import tpu_sc as plsc

B, N, H = 8, 1024, 256
D4 = 4 * H
M = 512
NB = (B * N) // M


def _dense_body(x_ref, c0_ref, h0_ref, c1_ref, h1_ref, ip_ref, keep_ref,
                ti_ref, fi_ref, wx0_ref, wh0_ref, b0_ref, wx1_ref, wh1_ref,
                b1_ref, k8_ref, kb_ref,
                emb_ref, wt_ref, wf_ref, rsum_ref, wsum_ref, ipsc_ref):
    x = x_ref[...]
    c0 = c0_ref[...]
    h0 = h0_ref[...]
    c1 = c1_ref[...]
    h1 = h1_ref[...]
    f32 = jnp.float32

    z0 = (jnp.dot(x, wx0_ref[...], preferred_element_type=f32)
          + jnp.dot(h0, wh0_ref[...], preferred_element_type=f32)
          + b0_ref[...])
    i0 = jax.nn.sigmoid(z0[:, 0:H])
    f0 = jax.nn.sigmoid(z0[:, H:2 * H])
    g0 = jnp.tanh(z0[:, 2 * H:3 * H])
    o0 = jax.nn.sigmoid(z0[:, 3 * H:4 * H])
    c0n = f0 * c0 + i0 * g0
    h0n = o0 * jnp.tanh(c0n)

    z1 = (jnp.dot(h0n, wx1_ref[...], preferred_element_type=f32)
          + jnp.dot(h1, wh1_ref[...], preferred_element_type=f32)
          + b1_ref[...])
    i1 = jax.nn.sigmoid(z1[:, 0:H])
    f1 = jax.nn.sigmoid(z1[:, H:2 * H])
    g1 = jnp.tanh(z1[:, 2 * H:3 * H])
    o1 = jax.nn.sigmoid(z1[:, 3 * H:4 * H])
    c1n = f1 * c1 + i1 * g1
    h1n = o1 * jnp.tanh(c1n)

    keep = keep_ref[...]
    c0n = keep * c0n + (1.0 - keep) * c0
    h0n = keep * h0n + (1.0 - keep) * h0
    c1n = keep * c1n + (1.0 - keep) * c1
    h1n = keep * h1n + (1.0 - keep) * h1

    emb = jnp.concatenate([c0n, h0n, c1n, h1n], axis=1)

    lg = jnp.dot(emb, k8_ref[...], preferred_element_type=f32) + kb_ref[...]
    r0 = lg[:, 0:1]
    r1 = lg[:, 1:2]
    mr = jnp.maximum(r0, r1)
    e0 = jnp.exp(r0 - mr)
    e1 = jnp.exp(r1 - mr)
    sr = e0 + e1
    p_raise = e0 / sr
    p_nor = e1 / sr
    q0 = lg[:, 2:3]
    q1 = lg[:, 3:4]
    mq = jnp.maximum(q0, q1)
    d0 = jnp.exp(q0 - mq)
    d1 = jnp.exp(q1 - mq)
    sq = d0 + d1
    pb0 = d0 / sq
    pb1 = d1 / sq

    ip = ip_ref[...]
    wr = p_raise * ip
    wt = p_nor * pb0 * ip
    wf = p_nor * pb1 * ip

    emb_ref[...] = emb
    wt_ref[...] = wt
    wf_ref[...] = wf

    rp = lax.dot_general(wr, emb, (((0,), (0,)), ((), ())),
                         preferred_element_type=f32)
    wsp = jnp.sum(wr)
    dcol = lax.broadcasted_iota(jnp.int32, (M, D4), 1)
    tb = ti_ref[...]
    fb = fi_ref[...]
    ipp = (jnp.sum(jnp.where(tb == dcol, wt, 0.0), axis=0, keepdims=True)
           + jnp.sum(jnp.where(fb == dcol, wf, 0.0), axis=0, keepdims=True))

    first = pl.program_id(0) % 2 == 0

    @pl.when(first)
    def _():
        rsum_ref[...] = rp[None]
        wsum_ref[...] = jnp.zeros((1, 1, 128), f32) + wsp
        ipsc_ref[...] = ipp[None]

    @pl.when(jnp.logical_not(first))
    def _():
        rsum_ref[...] += rp[None]
        wsum_ref[...] += wsp
        ipsc_ref[...] += ipp[None]


def _dense(xf, c0f, h0f, c1f, h1f, ipc, keep, tif, fif,
           Wx0, Wh0, b0r, Wx1, Wh1, b1r, K8, kb):
    f32 = jnp.float32
    row = lambda w: pl.BlockSpec((M, w), lambda i: (i, 0))
    full = lambda a, b: pl.BlockSpec((a, b), lambda i: (0, 0))
    acc = lambda w: pl.BlockSpec((1, 1, w), lambda i: (i // 2, 0, 0))
    return pl.pallas_call(
        _dense_body,
        grid=(NB,),
        in_specs=[row(H), row(H), row(H), row(H), row(H), row(1), row(1),
                  row(1), row(1), full(H, D4), full(H, D4), full(1, D4),
                  full(H, D4), full(H, D4), full(1, D4), full(D4, 8),
                  full(1, 8)],
        out_specs=[row(D4), row(1), row(1), acc(D4), acc(128), acc(D4)],
        out_shape=[jax.ShapeDtypeStruct((B * N, D4), f32),
                   jax.ShapeDtypeStruct((B * N, 1), f32),
                   jax.ShapeDtypeStruct((B * N, 1), f32),
                   jax.ShapeDtypeStruct((B, 1, D4), f32),
                   jax.ShapeDtypeStruct((B, 1, 128), f32),
                   jax.ShapeDtypeStruct((B, 1, D4), f32)],
    )(xf, c0f, h0f, c1f, h1f, ipc, keep, tif, fif,
      Wx0, Wh0, b0r, Wx1, Wh1, b1r, K8, kb)


_NC, _NS = 2, 16
_NW = _NC * _NS
_CW = 128
_CG = D4 // _CW
_NH = 2
_RH = N // _NH
_CH = 128
_CPR = N // _CH
_NT = B * _CG * _NH
_RNDS = _NT // _NW
_GC = _RNDS * _CPR


def _sc_body(emb_hbm, wt_hbm, wf_hbm, ti_hbm, fi_hbm, out_hbm,
             emA, tiA, fiA, wtA, wfA, emB, tiB, fiB, wtB, wfB,
             semA, semB, accum):
    c = lax.axis_index("c")
    s = lax.axis_index("s")
    wid = s * _NC + c
    banks = ((emA, tiA, fiA, wtA, wfA, semA), (emB, tiB, fiB, wtB, wfB, semB))

    def task(r):
        t = r * _NW + wid
        b = t // (_CG * _NH)
        rem = t % (_CG * _NH)
        g = rem // _NH
        h = rem % _NH
        return b * N, g * _CW, h * _RH

    def descs(gc, bnk):
        r = gc // _CPR
        ci = gc % _CPR
        row_base, col0, _ = task(r)
        row0 = row_base + ci * _CH
        emb_b, tib, fib, wtb, wfb, sem = banks[bnk]
        return (
            pltpu.make_async_copy(ti_hbm.at[pl.ds(row0, _CH)], tib, sem),
            pltpu.make_async_copy(fi_hbm.at[pl.ds(row0, _CH)], fib, sem),
            pltpu.make_async_copy(wt_hbm.at[pl.ds(row0, _CH)], wtb, sem),
            pltpu.make_async_copy(wf_hbm.at[pl.ds(row0, _CH)], wfb, sem),
            pltpu.make_async_copy(
                emb_hbm.at[pl.ds(row0, _CH), pl.ds(col0, _CW)], emb_b, sem),
        )

    def fire(gc, bnk):
        for d in descs(gc, bnk):
            d.start()

    def wait(gc, bnk):
        for d in descs(gc, bnk):
            d.wait()

    def zero_accum():
        zv = jnp.zeros((16,), jnp.float32)

        def zbody(i, cy):
            for j in range(_CW // 16):
                accum[i, pl.ds(j * 16, 16)] = zv
            return cy

        lax.fori_loop(0, _RH, zbody, 0)

    fire(0, 0)
    zero_accum()

    def pair_body(gg, carry0):
        for bnk in range(2):
            g = gg * 2 + bnk
            _, _, lo = task(g // _CPR)

            @pl.when(g + 1 < _GC)
            def _():
                fire(g + 1, bnk ^ 1)

            wait(g, bnk)
            emb_v, ti_v, fi_v, wt_v, wf_v, _sem = banks[bnk]

            def group_body(gi, carry2):
                sl16 = pl.ds(gi * 16, 16)
                tv = ti_v[sl16]
                fv = fi_v[sl16]
                wtv = wt_v[sl16]
                wfv = wf_v[sl16]
                for k in range(16):
                    row = gi * 16 + k
                    dt = tv[k]
                    df = fv[k]
                    lt = dt - lo
                    lf = df - lo
                    t_in = jnp.logical_and(dt >= lo, dt < lo + _RH)
                    f_in = jnp.logical_and(df >= lo, df < lo + _RH)

                    @pl.when(jnp.logical_or(t_in, f_in))
                    def _():
                        nj = _CW // 16
                        sls = [pl.ds(j * 16, 16) for j in range(nj)]
                        emv = [emb_v[row, sls[j]] for j in range(nj)]

                        @pl.when(t_in)
                        def _():
                            wv = jnp.full((16,), wtv[k], jnp.float32)
                            acc = [accum[lt, sls[j]] for j in range(nj)]
                            for j in range(nj):
                                accum[lt, sls[j]] = acc[j] + wv * emv[j]

                        @pl.when(f_in)
                        def _():
                            wv = jnp.full((16,), wfv[k], jnp.float32)
                            acc = [accum[lf, sls[j]] for j in range(nj)]
                            for j in range(nj):
                                accum[lf, sls[j]] = acc[j] + wv * emv[j]
                return carry2

            lax.fori_loop(0, _CH // 16, group_body, 0)

            @pl.when((g % _CPR) == (_CPR - 1))
            def _():
                r = g // _CPR
                row_base, col0, lo2 = task(r)
                pltpu.sync_copy(accum,
                                out_hbm.at[pl.ds(row_base + lo2, _RH),
                                           pl.ds(col0, _CW)])
                zero_accum()
        return carry0

    lax.fori_loop(0, _GC // 2, pair_body, 0)


@functools.lru_cache(maxsize=1)
def _build_sc_scatter():
    return functools.partial(
        pl.kernel,
        out_type=jax.ShapeDtypeStruct((B * N, D4), jnp.float32),
        mesh=plsc.VectorSubcoreMesh(core_axis_name="c", subcore_axis_name="s"),
        scratch_types=[
            pltpu.VMEM((_CH, _CW), jnp.float32),
            pltpu.VMEM((_CH,), jnp.int32),
            pltpu.VMEM((_CH,), jnp.int32),
            pltpu.VMEM((_CH,), jnp.float32),
            pltpu.VMEM((_CH,), jnp.float32),
            pltpu.VMEM((_CH, _CW), jnp.float32),
            pltpu.VMEM((_CH,), jnp.int32),
            pltpu.VMEM((_CH,), jnp.int32),
            pltpu.VMEM((_CH,), jnp.float32),
            pltpu.VMEM((_CH,), jnp.float32),
            pltpu.SemaphoreType.DMA,
            pltpu.SemaphoreType.DMA,
            pltpu.VMEM((_RH, _CW), jnp.float32),
        ],
    )(_sc_body)


def _final_body(agg_ref, c0_ref, h0_ref, c1_ref, h1_ref, ipsc_ref, ipold_ref,
                live_ref, exf_ref, rsum_ref, wsum_ref, hid_ref, ipo_ref):
    exf = exf_ref[...]
    agg = agg_ref[...] + exf * rsum_ref[0]
    w0 = wsum_ref[0][0:1, 0:1]
    ipn = ipsc_ref[...] + exf * w0
    denom = ipn + 1e-7
    live = live_ref[...]
    cat_old = jnp.concatenate(
        [c0_ref[...], h0_ref[...], c1_ref[...], h1_ref[...]], axis=1)
    hid_ref[...] = live * (agg / denom) + (1.0 - live) * cat_old
    ipo_ref[...] = live * ipn + (1.0 - live) * ipold_ref[...]


def _final(agg, c0f, h0f, c1f, h1f, ipscf, ipold, livef, exf, rsum, wsum):
    f32 = jnp.float32
    row = lambda w: pl.BlockSpec((M, w), lambda i: (i, 0))
    per_b = lambda w: pl.BlockSpec((1, 1, w), lambda i: (i // 2, 0, 0))
    return pl.pallas_call(
        _final_body,
        grid=(NB,),
        in_specs=[row(D4), row(H), row(H), row(H), row(H), row(1), row(1),
                  row(1), row(1), per_b(D4), per_b(128)],
        out_specs=[row(D4), row(1)],
        out_shape=[jax.ShapeDtypeStruct((B * N, D4), f32),
                   jax.ShapeDtypeStruct((B * N, 1), f32)],
    )(agg, c0f, h0f, c1f, h1f, ipscf, ipold, livef, exf, rsum, wsum)


def kernel(hidden_c0, hidden_h0, hidden_c1, hidden_h1, instruction_pointer,
           current_step, node_embeddings, Wx0, Wh0, b0, Wx1, Wh1, b1,
           raise_kernel, raise_bias, branch_kernel, branch_bias,
           edge_sources, edge_dests, edge_types, true_indexes, false_indexes,
           exit_indexes, step_limits):
    f32 = jnp.float32
    xf = node_embeddings.reshape(B * N, H)
    c0f = hidden_c0.reshape(B * N, H)
    h0f = hidden_h0.reshape(B * N, H)
    c1f = hidden_c1.reshape(B * N, H)
    h1f = hidden_h1.reshape(B * N, H)
    ipc = instruction_pointer.reshape(B * N, 1)
    tif = true_indexes.reshape(B * N, 1)
    fif = false_indexes.reshape(B * N, 1)

    node_ids = jnp.arange(N, dtype=jnp.int32)[None, :]
    is_exit = node_ids == exit_indexes[:, None].astype(jnp.int32)
    keep = jnp.where(is_exit, 0.0, 1.0).astype(f32).reshape(B * N, 1)
    exf = jnp.where(is_exit, 1.0, 0.0).astype(f32).reshape(B * N, 1)
    live = (current_step < step_limits).astype(f32)
    livef = jnp.broadcast_to(live[:, None], (B, N)).reshape(B * N, 1)

    K8 = jnp.concatenate(
        [raise_kernel, branch_kernel, jnp.zeros((D4, 4), f32)], axis=1)
    kb = jnp.concatenate(
        [raise_bias, branch_bias, jnp.zeros((4,), f32)])[None, :]
    b0r = b0[None, :]
    b1r = b1[None, :]

    emb, wt2, wf2, rsum, wsum, ipsc = _dense(
        xf, c0f, h0f, c1f, h1f, ipc, keep, tif, fif,
        Wx0, Wh0, b0r, Wx1, Wh1, b1r, K8, kb)

    agg = _build_sc_scatter()(emb, wt2.reshape(B * N), wf2.reshape(B * N),
                              true_indexes.reshape(B * N),
                              false_indexes.reshape(B * N))

    ipscf = ipsc.reshape(B * N, 1)
    hid, ipo = _final(agg, c0f, h0f, c1f, h1f, ipscf, ipc, livef, exf,
                      rsum, wsum)
    return hid.reshape(B, N, D4), ipo.reshape(B, N)
